# baseline (speedup 1.0000x reference)
import functools

import jax
import jax.numpy as jnp
from jax.experimental import pallas as pl
from jax.experimental.pallas import tpu as pltpu


def _body(x0_ref, x1_ref, w_ref, b_ref, out_ref, lacc_ref, *, nsteps, s_total):
    b = pl.program_id(0)
    j = pl.program_id(1)

    @pl.when(jnp.logical_and(b == 0, j == 0))
    def _init():
        lacc_ref[...] = jnp.zeros_like(lacc_ref)

    p0 = jnp.sum(x0_ref[...], axis=1)
    p1 = jnp.sum(x1_ref[...], axis=1)
    part = jnp.concatenate([p0, p1], axis=1)
    lacc_ref[pl.ds(b, 1), :] += jax.lax.dot_general(
        part, w_ref[...],
        dimension_numbers=(((1,), (1,)), ((), ())),
        preferred_element_type=jnp.float32,
    )

    @pl.when(jnp.logical_and(b == pl.num_programs(0) - 1, j == nsteps - 1))
    def _finish():
        logits = lacc_ref[...] * (1.0 / s_total) + b_ref[...]
        m = jnp.max(logits, axis=-1, keepdims=True)
        e = jnp.exp(logits - m)
        out_ref[...] = e / jnp.sum(e, axis=-1, keepdims=True)


def kernel(x, gate_weight, gate_bias):
    B, S, D = x.shape
    M = gate_weight.shape[0]
    s_blk = 1024
    while S % s_blk != 0:
        s_blk //= 2
    nsteps = S // s_blk
    hd = D // 2

    bias2d = gate_bias.reshape(1, M)

    return pl.pallas_call(
        functools.partial(_body, nsteps=nsteps, s_total=S),
        grid=(B, nsteps),
        in_specs=[
            pl.BlockSpec((1, s_blk, hd), lambda b, j: (b, j, 0)),
            pl.BlockSpec((1, s_blk, hd), lambda b, j: (b, j, 1)),
            pl.BlockSpec((M, D), lambda b, j: (0, 0)),
            pl.BlockSpec((1, M), lambda b, j: (0, 0)),
        ],
        out_specs=pl.BlockSpec((B, M), lambda b, j: (0, 0)),
        out_shape=jax.ShapeDtypeStruct((B, M), jnp.float32),
        scratch_shapes=[pltpu.VMEM((B, M), jnp.float32)],
    )(x, x, gate_weight, bias2d)
